# baseline (speedup 1.0000x reference)
import functools

import jax
import jax.numpy as jnp
from jax import lax
from jax.experimental import pallas as pl
from jax.experimental.pallas import tpu as pltpu
from jax.experimental.pallas import tpu_sc as plsc

MODEL_DIM = 256
FEATURE_DIM = 32
GRID_SIZE = 16
NUM_HEADS = 8
BATCH = 4
NPTS = 8192
C_OUT = NUM_HEADS * (FEATURE_DIM + 3)
G3 = GRID_SIZE ** 3
NPAIR = BATCH * NUM_HEADS

PT = 2048
NSTEP = NPTS // PT

SC_CORES = 2
SC_SUBCORES = 16

SC_CHUNK = 256
N_CHUNK = NPTS // SC_CHUNK
FHALF = 16
N_ROUND = FEATURE_DIM // FHALF
TBL = G3 * FHALF
VT_STRIDE = FHALF + 1

_CORNER_OFF = [
    (bx * GRID_SIZE * GRID_SIZE + by * GRID_SIZE + bz)
    for bx in (0, 1) for by in (0, 1) for bz in (0, 1)
]
_OFF16 = [off * FHALF for off in _CORNER_OFF]


def _stats_body(x_ref, w_ref, s1_ref, s2_ref):
    b = pl.program_id(0)
    i = pl.program_id(1)

    @pl.when(jnp.logical_and(b == 0, i == 0))
    def _():
        s1_ref[...] = jnp.zeros_like(s1_ref)
        s2_ref[...] = jnp.zeros_like(s2_ref)

    kv = jnp.dot(w_ref[...], x_ref[0], preferred_element_type=jnp.float32)
    s1_ref[...] += jnp.sum(kv, axis=1, keepdims=True)
    s2_ref[...] += jnp.sum(kv * kv, axis=1, keepdims=True)


def _geom_body(x_ref, o_ref, w_ref, s1_ref, s2_ref, gv_ref, bv_ref, gk_ref,
               bk_ref, m24_ref, s8_ref, vals_ref, w8_ref, base_ref,
               ksum_ref, ksq_ref):
    b = pl.program_id(0)
    i = pl.program_id(1)

    @pl.when(jnp.logical_and(b == 0, i == 0))
    def _():
        ksum_ref[...] = jnp.zeros_like(ksum_ref)
        ksq_ref[...] = jnp.zeros_like(ksq_ref)

    n = float(BATCH * NPTS)
    kv = jnp.dot(w_ref[...], x_ref[0], preferred_element_type=jnp.float32)

    mean = s1_ref[...] / n
    var = s2_ref[...] / n - mean * mean
    rstd = lax.rsqrt(var + 1e-5)

    ko = kv[0:NUM_HEADS * 3]
    vals = kv[NUM_HEADS * 3:]
    ko_n = (ko - mean[0:24]) * (rstd[0:24] * gk_ref[...]) + bk_ref[...]
    vals_ref[0] = ((vals - mean[24:]) * (rstd[24:] * gv_ref[...])
                   + bv_ref[...])

    orig = o_ref[0]
    pts = ko_n + jnp.concatenate([orig] * NUM_HEADS, axis=0)
    keys = jnp.dot(m24_ref[...], pts, preferred_element_type=jnp.float32)

    ksum_ref[...] += jnp.full((1, 128), jnp.sum(keys), jnp.float32)
    ksq_ref[...] += jnp.full((1, 128), jnp.sum(keys * keys), jnp.float32)

    lattice = jnp.tanh(keys)
    pos = (lattice + 1.0) * 0.5 * float(GRID_SIZE - 1)
    basef = jnp.clip(jnp.floor(pos), 0.0, float(GRID_SIZE - 2))
    frac = pos - basef
    flat16 = jnp.dot(s8_ref[...], basef, preferred_element_type=jnp.float32)
    base_ref[0] = flat16.astype(jnp.int32)

    fx = frac[0:NUM_HEADS]
    fy = frac[NUM_HEADS:2 * NUM_HEADS]
    fz = frac[2 * NUM_HEADS:3 * NUM_HEADS]
    gx = 1.0 - fx
    gy = 1.0 - fy
    gz = 1.0 - fz
    w00 = gx * gy
    w01 = gx * fy
    w10 = fx * gy
    w11 = fx * fy
    wc = (w00 * gz, w00 * fz, w01 * gz, w01 * fz,
          w10 * gz, w10 * fz, w11 * gz, w11 * fz)
    w8_ref[0] = jnp.stack(wc, axis=1)


def _splat_sc_body(vals_hbm, w8_hbm, base_hbm, zeros_hbm, out_hbm,
                   table_v, vt_v, vals_a, w_a, b_a, vals_b, w_b, b_b,
                   sem_a, sem_b):
    wid = lax.axis_index("s") * SC_CORES + lax.axis_index("c")
    pair = wid
    lanes17 = jnp.arange(16, dtype=jnp.int32) * VT_STRIDE

    def _copies(r, ci, vv, wv, bv, sem):
        c0 = ci * SC_CHUNK
        return (
            pltpu.make_async_copy(
                vals_hbm.at[pair, pl.ds(r * FHALF, FHALF),
                            pl.ds(c0, SC_CHUNK)], vv, sem),
            pltpu.make_async_copy(
                w8_hbm.at[pair, :, pl.ds(c0, SC_CHUNK)], wv, sem),
            pltpu.make_async_copy(
                base_hbm.at[pair, pl.ds(c0, SC_CHUNK)], bv, sem),
        )

    def _start(r, ci, vv, wv, bv, sem):
        for d in _copies(r, ci, vv, wv, bv, sem):
            d.start()

    def _wait(r, ci, vv, wv, bv, sem):
        for d in _copies(r, ci, vv, wv, bv, sem):
            d.wait()

    def _scatter_chunk(vv, wv, bv):
        def tr_body(b4, _):
            for u in range(4):
                b0 = (b4 * 4 + u) * 16
                off = b0 * VT_STRIDE
                for f in range(FHALF):
                    vec = vv[f, pl.ds(b0, 16)]
                    plsc.store_scatter(vt_v, [lanes17 + (off + f)], vec)
            return 0

        lax.fori_loop(0, SC_CHUNK // 64, tr_body, 0)

        def sup_body(s, _):
            s0 = s * 16
            bvec = bv[pl.ds(s0, 16)]
            wvecs = [wv[c, pl.ds(s0, 16)] for c in range(8)]
            for p in range(16):
                vp = vt_v[pl.ds((s0 + p) * VT_STRIDE, FHALF)]
                bse = bvec[p]
                for c in range(8):
                    wb = jnp.full((16,), wvecs[c][p], jnp.float32)
                    plsc.addupdate(table_v.at[pl.ds(bse + _OFF16[c], FHALF)],
                                   wb * vp)
            return 0

        lax.fori_loop(0, SC_CHUNK // 16, sup_body, 0)

    for r in range(N_ROUND):
        pltpu.sync_copy(zeros_hbm, table_v)
        _start(r, 0, vals_a, w_a, b_a, sem_a)
        _start(r, 1, vals_b, w_b, b_b, sem_b)

        def pair_body(i2, _, r=r):
            ca = 2 * i2
            _wait(r, ca, vals_a, w_a, b_a, sem_a)
            _scatter_chunk(vals_a, w_a, b_a)

            @pl.when(i2 < N_CHUNK // 2 - 1)
            def _():
                _start(r, ca + 2, vals_a, w_a, b_a, sem_a)

            _wait(r, ca + 1, vals_b, w_b, b_b, sem_b)
            _scatter_chunk(vals_b, w_b, b_b)

            @pl.when(i2 < N_CHUNK // 2 - 1)
            def _():
                _start(r, ca + 3, vals_b, w_b, b_b, sem_b)

            return 0

        lax.fori_loop(0, N_CHUNK // 2, pair_body, 0)
        pltpu.sync_copy(table_v,
                        out_hbm.at[pl.ds((pair * N_ROUND + r) * TBL, TBL)])


def _finalize_body(splat_ref, ksum_ref, ksq_ref, occ_ref, km_ref, kv_ref,
                   acc_ref):
    i = pl.program_id(0)

    @pl.when(i == 0)
    def _():
        acc_ref[...] = jnp.zeros_like(acc_ref)

    x = splat_ref[0]
    nz = (jnp.abs(x) > 1e-9).astype(jnp.float32)
    acc_ref[...] += jnp.full((1, 128), jnp.sum(nz), jnp.float32)

    @pl.when(i == pl.num_programs(0) - 1)
    def _():
        occ = acc_ref[0, 0] / float(BATCH * FEATURE_DIM * NUM_HEADS)
        nk = float(BATCH * NUM_HEADS * 3 * NPTS)
        kmean = ksum_ref[0, 0] / nk
        kvar = (ksq_ref[0, 0] - nk * kmean * kmean) / (nk - 1.0)
        occ_ref[...] = jnp.full((1, 1), occ, jnp.float32)
        km_ref[...] = jnp.full((1, 1), kmean, jnp.float32)
        kv_ref[...] = jnp.full((1, 1), kvar, jnp.float32)


@functools.lru_cache(maxsize=1)
def _get_splat_sc():
    return pl.kernel(
        _splat_sc_body,
        out_type=jax.ShapeDtypeStruct((NPAIR * N_ROUND * TBL,), jnp.float32),
        mesh=plsc.VectorSubcoreMesh(core_axis_name="c", subcore_axis_name="s"),
        compiler_params=pltpu.CompilerParams(needs_layout_passes=False),
        scratch_types=[
            pltpu.VMEM((TBL,), jnp.float32),
            pltpu.VMEM((SC_CHUNK * VT_STRIDE,), jnp.float32),
            pltpu.VMEM((FHALF, SC_CHUNK), jnp.float32),
            pltpu.VMEM((8, SC_CHUNK), jnp.float32),
            pltpu.VMEM((SC_CHUNK,), jnp.int32),
            pltpu.VMEM((FHALF, SC_CHUNK), jnp.float32),
            pltpu.VMEM((8, SC_CHUNK), jnp.float32),
            pltpu.VMEM((SC_CHUNK,), jnp.int32),
            pltpu.SemaphoreType.DMA,
            pltpu.SemaphoreType.DMA,
        ],
    )


def kernel(input_tensor, original_points, W_kv, values_gamma, values_beta,
           key_gamma, key_beta, rot):
    f32 = jnp.float32

    s1, s2 = pl.pallas_call(
        _stats_body,
        grid=(BATCH, NSTEP),
        in_specs=[
            pl.BlockSpec((1, MODEL_DIM, PT), lambda b, i: (b, 0, i)),
            pl.BlockSpec((C_OUT, MODEL_DIM), lambda b, i: (0, 0)),
        ],
        out_specs=[
            pl.BlockSpec((C_OUT, 1), lambda b, i: (0, 0)),
            pl.BlockSpec((C_OUT, 1), lambda b, i: (0, 0)),
        ],
        out_shape=[
            jax.ShapeDtypeStruct((C_OUT, 1), f32),
            jax.ShapeDtypeStruct((C_OUT, 1), f32),
        ],
    )(input_tensor, W_kv)

    m24p = jnp.zeros((24, 24), f32)
    ii, hh, jj = jnp.meshgrid(jnp.arange(3), jnp.arange(NUM_HEADS),
                              jnp.arange(3), indexing="ij")
    m24p = m24p.at[ii * NUM_HEADS + hh, hh * 3 + jj].set(
        rot.astype(f32).transpose(1, 0, 2)[ii, hh, jj])
    s8p16 = jnp.zeros((NUM_HEADS, 24), f32)
    scales16 = 16.0 * jnp.array([GRID_SIZE * GRID_SIZE, GRID_SIZE, 1], f32)
    hs = jnp.tile(jnp.arange(NUM_HEADS), 3)
    isx = jnp.repeat(jnp.arange(3), NUM_HEADS)
    s8p16 = s8p16.at[hs, isx * NUM_HEADS + hs].set(
        jnp.repeat(scales16, NUM_HEADS))

    vals_n, w8, base16, ksum, ksq = pl.pallas_call(
        _geom_body,
        grid=(BATCH, NSTEP),
        in_specs=[
            pl.BlockSpec((1, MODEL_DIM, PT), lambda b, i: (b, 0, i)),
            pl.BlockSpec((1, 3, PT), lambda b, i: (b, 0, i)),
            pl.BlockSpec((C_OUT, MODEL_DIM), lambda b, i: (0, 0)),
            pl.BlockSpec((C_OUT, 1), lambda b, i: (0, 0)),
            pl.BlockSpec((C_OUT, 1), lambda b, i: (0, 0)),
            pl.BlockSpec((NUM_HEADS * FEATURE_DIM, 1), lambda b, i: (0, 0)),
            pl.BlockSpec((NUM_HEADS * FEATURE_DIM, 1), lambda b, i: (0, 0)),
            pl.BlockSpec((NUM_HEADS * 3, 1), lambda b, i: (0, 0)),
            pl.BlockSpec((NUM_HEADS * 3, 1), lambda b, i: (0, 0)),
            pl.BlockSpec((24, 24), lambda b, i: (0, 0)),
            pl.BlockSpec((NUM_HEADS, 24), lambda b, i: (0, 0)),
        ],
        out_specs=[
            pl.BlockSpec((1, NUM_HEADS * FEATURE_DIM, PT),
                         lambda b, i: (b, 0, i)),
            pl.BlockSpec((1, NUM_HEADS, 8, PT), lambda b, i: (b, 0, 0, i)),
            pl.BlockSpec((1, NUM_HEADS, PT), lambda b, i: (b, 0, i)),
            pl.BlockSpec((1, 128), lambda b, i: (0, 0)),
            pl.BlockSpec((1, 128), lambda b, i: (0, 0)),
        ],
        out_shape=[
            jax.ShapeDtypeStruct((BATCH, NUM_HEADS * FEATURE_DIM, NPTS), f32),
            jax.ShapeDtypeStruct((BATCH, NUM_HEADS, 8, NPTS), f32),
            jax.ShapeDtypeStruct((BATCH, NUM_HEADS, NPTS), jnp.int32),
            jax.ShapeDtypeStruct((1, 128), f32),
            jax.ShapeDtypeStruct((1, 128), f32),
        ],
    )(input_tensor, original_points, W_kv, s1, s2,
      values_gamma.reshape(-1, 1), values_beta.reshape(-1, 1),
      key_gamma.reshape(-1, 1), key_beta.reshape(-1, 1), m24p, s8p16)

    zeros_tbl = jnp.zeros((TBL,), f32)
    splat_flat = _get_splat_sc()(
        vals_n.reshape(NPAIR, FEATURE_DIM, NPTS),
        w8.reshape(NPAIR, 8, NPTS),
        base16.reshape(NPAIR, NPTS), zeros_tbl)

    splat_view = splat_flat.reshape(NPAIR * N_ROUND, 8, TBL // 8)
    occ, kmean, kvar = pl.pallas_call(
        _finalize_body,
        grid=(NPAIR * N_ROUND,),
        in_specs=[
            pl.BlockSpec((1, 8, TBL // 8), lambda i: (i, 0, 0)),
            pl.BlockSpec((1, 128), lambda i: (0, 0)),
            pl.BlockSpec((1, 128), lambda i: (0, 0)),
        ],
        out_specs=[
            pl.BlockSpec((1, 1), lambda i: (0, 0)),
            pl.BlockSpec((1, 1), lambda i: (0, 0)),
            pl.BlockSpec((1, 1), lambda i: (0, 0)),
        ],
        out_shape=[
            jax.ShapeDtypeStruct((1, 1), f32),
            jax.ShapeDtypeStruct((1, 1), f32),
            jax.ShapeDtypeStruct((1, 1), f32),
        ],
        scratch_shapes=[pltpu.VMEM((1, 128), f32)],
    )(splat_view, ksum, ksq)

    splat = splat_flat.reshape(NPAIR, N_ROUND, G3, FHALF)
    splat = jnp.transpose(splat, (0, 1, 3, 2))
    splat = splat.reshape(BATCH, NUM_HEADS * FEATURE_DIM,
                          GRID_SIZE, GRID_SIZE, GRID_SIZE)
    return (splat, occ[0, 0], kmean[0, 0], kvar[0, 0])


# speedup vs baseline: 1.1084x; 1.1084x over previous
import functools

import jax
import jax.numpy as jnp
from jax import lax
from jax.experimental import pallas as pl
from jax.experimental.pallas import tpu as pltpu
from jax.experimental.pallas import tpu_sc as plsc

MODEL_DIM = 256
FEATURE_DIM = 32
GRID_SIZE = 16
NUM_HEADS = 8
BATCH = 4
NPTS = 8192
C_OUT = NUM_HEADS * (FEATURE_DIM + 3)
G3 = GRID_SIZE ** 3
NPAIR = BATCH * NUM_HEADS

PT = 2048
NSTEP = NPTS // PT

SC_CORES = 2
SC_SUBCORES = 16
NW = SC_CORES * SC_SUBCORES

SC_CHUNK = 1024
N_CHUNK = NPTS // SC_CHUNK
FHALF = 16
N_ROUND = FEATURE_DIM // FHALF
TBL = FHALF * G3

_CORNER_OFF = [
    (bx * GRID_SIZE * GRID_SIZE + by * GRID_SIZE + bz)
    for bx in (0, 1) for by in (0, 1) for bz in (0, 1)
]


def _stats_body(x_ref, w_ref, s1_ref, s2_ref):
    b = pl.program_id(0)
    i = pl.program_id(1)

    @pl.when(jnp.logical_and(b == 0, i == 0))
    def _():
        s1_ref[...] = jnp.zeros_like(s1_ref)
        s2_ref[...] = jnp.zeros_like(s2_ref)

    kv = jnp.dot(w_ref[...], x_ref[0], preferred_element_type=jnp.float32)
    s1_ref[...] += jnp.sum(kv, axis=1, keepdims=True)
    s2_ref[...] += jnp.sum(kv * kv, axis=1, keepdims=True)


def _geom_body(x_ref, o_ref, w_ref, s1_ref, s2_ref, gv_ref, bv_ref, gk_ref,
               bk_ref, m24_ref, s8_ref, vals_ref, frac_ref, base_ref,
               ksum_ref, ksq_ref):
    b = pl.program_id(0)
    i = pl.program_id(1)

    @pl.when(jnp.logical_and(b == 0, i == 0))
    def _():
        ksum_ref[...] = jnp.zeros_like(ksum_ref)
        ksq_ref[...] = jnp.zeros_like(ksq_ref)

    n = float(BATCH * NPTS)
    kv = jnp.dot(w_ref[...], x_ref[0], preferred_element_type=jnp.float32)

    mean = s1_ref[...] / n
    var = s2_ref[...] / n - mean * mean
    rstd = lax.rsqrt(var + 1e-5)

    ko = kv[0:NUM_HEADS * 3]
    vals = kv[NUM_HEADS * 3:]
    ko_n = (ko - mean[0:24]) * (rstd[0:24] * gk_ref[...]) + bk_ref[...]
    vals_n = (vals - mean[24:]) * (rstd[24:] * gv_ref[...]) + bv_ref[...]
    vals_ref[0] = vals_n

    orig = o_ref[0]
    pts = ko_n + jnp.concatenate([orig] * NUM_HEADS, axis=0)
    keys = jnp.dot(m24_ref[...], pts, preferred_element_type=jnp.float32)

    ksum_ref[...] += jnp.full((1, 128), jnp.sum(keys), jnp.float32)
    ksq_ref[...] += jnp.full((1, 128), jnp.sum(keys * keys), jnp.float32)

    lattice = jnp.tanh(keys)
    pos = (lattice + 1.0) * 0.5 * float(GRID_SIZE - 1)
    basef = jnp.clip(jnp.floor(pos), 0.0, float(GRID_SIZE - 2))
    frac_ref[0] = pos - basef
    flat = jnp.dot(s8_ref[...], basef, preferred_element_type=jnp.float32)
    base_ref[0] = flat.astype(jnp.int32)


def _splat_sc_body(vals_hbm, frac_hbm, base_hbm, zeros_hbm, out_hbm,
                   table_v, vals_a, frac_a, base_a, vals_b, frac_b, base_b,
                   sem_a, sem_b):
    wid = lax.axis_index("s") * SC_CORES + lax.axis_index("c")
    pair = wid

    def _copies(r, ci, vv, fv, bv, sem):
        c0 = ci * SC_CHUNK
        return (
            pltpu.make_async_copy(
                vals_hbm.at[pair, pl.ds(r * FHALF, FHALF),
                            pl.ds(c0, SC_CHUNK)], vv, sem),
            pltpu.make_async_copy(
                frac_hbm.at[pair, :, pl.ds(c0, SC_CHUNK)], fv, sem),
            pltpu.make_async_copy(
                base_hbm.at[pair, pl.ds(c0, SC_CHUNK)], bv, sem),
        )

    def _start(r, ci, vv, fv, bv, sem):
        for d in _copies(r, ci, vv, fv, bv, sem):
            d.start()

    def _wait(r, ci, vv, fv, bv, sem):
        for d in _copies(r, ci, vv, fv, bv, sem):
            d.wait()

    def _scatter_chunk(vv, fv, bv):
        def group_body(g, _):
            g0 = g * 16
            fx = fv[0, pl.ds(g0, 16)]
            fy = fv[1, pl.ds(g0, 16)]
            fz = fv[2, pl.ds(g0, 16)]
            bse = bv[pl.ds(g0, 16)]
            gx = 1.0 - fx
            gy = 1.0 - fy
            gz = 1.0 - fz
            w00 = gx * gy
            w01 = gx * fy
            w10 = fx * gy
            w11 = fx * fy
            wc = (w00 * gz, w00 * fz, w01 * gz, w01 * fz,
                  w10 * gz, w10 * fz, w11 * gz, w11 * fz)
            cells = tuple(bse + off for off in _CORNER_OFF)
            for f in range(FHALF):
                v = vv[f, pl.ds(g0, 16)]
                fbase = f * G3
                for c in range(8):
                    plsc.addupdate_scatter(
                        table_v, [cells[c] + fbase], wc[c] * v)
            return 0

        lax.fori_loop(0, SC_CHUNK // 16, group_body, 0)

    for r in range(N_ROUND):
        pltpu.sync_copy(zeros_hbm, table_v)
        _start(r, 0, vals_a, frac_a, base_a, sem_a)
        _start(r, 1, vals_b, frac_b, base_b, sem_b)

        def pair_body(i2, _, r=r):
            ca = 2 * i2
            _wait(r, ca, vals_a, frac_a, base_a, sem_a)
            _scatter_chunk(vals_a, frac_a, base_a)

            @pl.when(i2 < N_CHUNK // 2 - 1)
            def _():
                _start(r, ca + 2, vals_a, frac_a, base_a, sem_a)

            _wait(r, ca + 1, vals_b, frac_b, base_b, sem_b)
            _scatter_chunk(vals_b, frac_b, base_b)

            @pl.when(i2 < N_CHUNK // 2 - 1)
            def _():
                _start(r, ca + 3, vals_b, frac_b, base_b, sem_b)

            return 0

        lax.fori_loop(0, N_CHUNK // 2, pair_body, 0)
        pltpu.sync_copy(table_v, out_hbm.at[pair, r])


def _finalize_body(splat_ref, ksum_ref, ksq_ref, occ_ref, km_ref, kv_ref,
                   acc_ref):
    i = pl.program_id(0)

    @pl.when(i == 0)
    def _():
        acc_ref[...] = jnp.zeros_like(acc_ref)

    x = splat_ref[0]
    nz = (jnp.abs(x) > 1e-9).astype(jnp.float32)
    acc_ref[...] += jnp.sum(nz, axis=0, keepdims=True)

    @pl.when(i == pl.num_programs(0) - 1)
    def _():
        occ = jnp.sum(acc_ref[...]) / float(BATCH * FEATURE_DIM * NUM_HEADS)
        nk = float(BATCH * NUM_HEADS * 3 * NPTS)
        kmean = ksum_ref[0, 0] / nk
        kvar = (ksq_ref[0, 0] - nk * kmean * kmean) / (nk - 1.0)
        occ_ref[...] = jnp.full((1, 1), occ, jnp.float32)
        km_ref[...] = jnp.full((1, 1), kmean, jnp.float32)
        kv_ref[...] = jnp.full((1, 1), kvar, jnp.float32)


@functools.lru_cache(maxsize=1)
def _get_splat_sc():
    return pl.kernel(
        _splat_sc_body,
        out_type=jax.ShapeDtypeStruct((NPAIR, N_ROUND, TBL), jnp.float32),
        mesh=plsc.VectorSubcoreMesh(core_axis_name="c", subcore_axis_name="s"),
        compiler_params=pltpu.CompilerParams(needs_layout_passes=False),
        scratch_types=[
            pltpu.VMEM((TBL,), jnp.float32),
            pltpu.VMEM((FHALF, SC_CHUNK), jnp.float32),
            pltpu.VMEM((3, SC_CHUNK), jnp.float32),
            pltpu.VMEM((SC_CHUNK,), jnp.int32),
            pltpu.VMEM((FHALF, SC_CHUNK), jnp.float32),
            pltpu.VMEM((3, SC_CHUNK), jnp.float32),
            pltpu.VMEM((SC_CHUNK,), jnp.int32),
            pltpu.SemaphoreType.DMA,
            pltpu.SemaphoreType.DMA,
        ],
    )


def kernel(input_tensor, original_points, W_kv, values_gamma, values_beta,
           key_gamma, key_beta, rot):
    f32 = jnp.float32

    s1, s2 = pl.pallas_call(
        _stats_body,
        grid=(BATCH, NSTEP),
        in_specs=[
            pl.BlockSpec((1, MODEL_DIM, PT), lambda b, i: (b, 0, i)),
            pl.BlockSpec((C_OUT, MODEL_DIM), lambda b, i: (0, 0)),
        ],
        out_specs=[
            pl.BlockSpec((C_OUT, 1), lambda b, i: (0, 0)),
            pl.BlockSpec((C_OUT, 1), lambda b, i: (0, 0)),
        ],
        out_shape=[
            jax.ShapeDtypeStruct((C_OUT, 1), f32),
            jax.ShapeDtypeStruct((C_OUT, 1), f32),
        ],
    )(input_tensor, W_kv)

    m24 = jax.scipy.linalg.block_diag(*[rot[h] for h in range(NUM_HEADS)])
    m24 = m24.astype(f32)
    s8 = jnp.zeros((NUM_HEADS, NUM_HEADS * 3), f32)
    scales = jnp.array([GRID_SIZE * GRID_SIZE, GRID_SIZE, 1], f32)
    rows = jnp.repeat(jnp.arange(NUM_HEADS), 3)
    cols = jnp.arange(NUM_HEADS * 3)
    s8 = s8.at[rows, cols].set(jnp.tile(scales, NUM_HEADS))

    vals_n, frac24, basef, ksum, ksq = pl.pallas_call(
        _geom_body,
        grid=(BATCH, NSTEP),
        in_specs=[
            pl.BlockSpec((1, MODEL_DIM, PT), lambda b, i: (b, 0, i)),
            pl.BlockSpec((1, 3, PT), lambda b, i: (b, 0, i)),
            pl.BlockSpec((C_OUT, MODEL_DIM), lambda b, i: (0, 0)),
            pl.BlockSpec((C_OUT, 1), lambda b, i: (0, 0)),
            pl.BlockSpec((C_OUT, 1), lambda b, i: (0, 0)),
            pl.BlockSpec((NUM_HEADS * FEATURE_DIM, 1), lambda b, i: (0, 0)),
            pl.BlockSpec((NUM_HEADS * FEATURE_DIM, 1), lambda b, i: (0, 0)),
            pl.BlockSpec((NUM_HEADS * 3, 1), lambda b, i: (0, 0)),
            pl.BlockSpec((NUM_HEADS * 3, 1), lambda b, i: (0, 0)),
            pl.BlockSpec((24, 24), lambda b, i: (0, 0)),
            pl.BlockSpec((NUM_HEADS, 24), lambda b, i: (0, 0)),
        ],
        out_specs=[
            pl.BlockSpec((1, NUM_HEADS * FEATURE_DIM, PT), lambda b, i: (b, 0, i)),
            pl.BlockSpec((1, NUM_HEADS * 3, PT), lambda b, i: (b, 0, i)),
            pl.BlockSpec((1, NUM_HEADS, PT), lambda b, i: (b, 0, i)),
            pl.BlockSpec((1, 128), lambda b, i: (0, 0)),
            pl.BlockSpec((1, 128), lambda b, i: (0, 0)),
        ],
        out_shape=[
            jax.ShapeDtypeStruct((BATCH, NUM_HEADS * FEATURE_DIM, NPTS), f32),
            jax.ShapeDtypeStruct((BATCH, NUM_HEADS * 3, NPTS), f32),
            jax.ShapeDtypeStruct((BATCH, NUM_HEADS, NPTS), jnp.int32),
            jax.ShapeDtypeStruct((1, 128), f32),
            jax.ShapeDtypeStruct((1, 128), f32),
        ],
    )(input_tensor, original_points, W_kv, s1, s2,
      values_gamma.reshape(-1, 1), values_beta.reshape(-1, 1),
      key_gamma.reshape(-1, 1), key_beta.reshape(-1, 1), m24, s8)

    vals_sc = vals_n.reshape(NPAIR, FEATURE_DIM, NPTS)
    frac_sc = frac24.reshape(NPAIR, 3, NPTS)
    base_sc = basef.reshape(NPAIR, NPTS)
    zeros_tbl = jnp.zeros((TBL,), f32)
    splat_flat = _get_splat_sc()(vals_sc, frac_sc, base_sc, zeros_tbl)

    splat_view = splat_flat.reshape(NPAIR * N_ROUND, TBL // 128, 128)
    occ, kmean, kvar = pl.pallas_call(
        _finalize_body,
        grid=(NPAIR * N_ROUND,),
        in_specs=[
            pl.BlockSpec((1, TBL // 128, 128), lambda i: (i, 0, 0)),
            pl.BlockSpec((1, 128), lambda i: (0, 0)),
            pl.BlockSpec((1, 128), lambda i: (0, 0)),
        ],
        out_specs=[
            pl.BlockSpec((1, 1), lambda i: (0, 0)),
            pl.BlockSpec((1, 1), lambda i: (0, 0)),
            pl.BlockSpec((1, 1), lambda i: (0, 0)),
        ],
        out_shape=[
            jax.ShapeDtypeStruct((1, 1), f32),
            jax.ShapeDtypeStruct((1, 1), f32),
            jax.ShapeDtypeStruct((1, 1), f32),
        ],
        scratch_shapes=[pltpu.VMEM((1, 128), f32)],
    )(splat_view, ksum, ksq)

    splat = splat_flat.reshape(BATCH, NUM_HEADS * FEATURE_DIM,
                               GRID_SIZE, GRID_SIZE, GRID_SIZE)
    return (splat, occ[0, 0], kmean[0, 0], kvar[0, 0])


# speedup vs baseline: 1.1263x; 1.0161x over previous
import functools

import jax
import jax.numpy as jnp
from jax import lax
from jax.experimental import pallas as pl
from jax.experimental.pallas import tpu as pltpu
from jax.experimental.pallas import tpu_sc as plsc

MODEL_DIM = 256
FEATURE_DIM = 32
GRID_SIZE = 16
NUM_HEADS = 8
BATCH = 4
NPTS = 8192
C_OUT = NUM_HEADS * (FEATURE_DIM + 3)
G3 = GRID_SIZE ** 3
NPAIR = BATCH * NUM_HEADS

PT = 4096
NSTEP = NPTS // PT

SC_CORES = 2
SC_SUBCORES = 16
NW = SC_CORES * SC_SUBCORES

SC_CHUNK = 1024
N_CHUNK = NPTS // SC_CHUNK
FHALF = 16
N_ROUND = FEATURE_DIM // FHALF
TBL = FHALF * G3

_CORNER_OFF = [
    (bx * GRID_SIZE * GRID_SIZE + by * GRID_SIZE + bz)
    for bx in (0, 1) for by in (0, 1) for bz in (0, 1)
]


def _stats_body(x_ref, w_ref, s1_ref, s2_ref):
    b = pl.program_id(0)
    i = pl.program_id(1)

    @pl.when(jnp.logical_and(b == 0, i == 0))
    def _():
        s1_ref[...] = jnp.zeros_like(s1_ref)
        s2_ref[...] = jnp.zeros_like(s2_ref)

    kv = jnp.dot(w_ref[...], x_ref[0], preferred_element_type=jnp.float32)
    s1_ref[...] += jnp.sum(kv, axis=1, keepdims=True)
    s2_ref[...] += jnp.sum(kv * kv, axis=1, keepdims=True)


def _geom_body(x_ref, o_ref, w_ref, s1_ref, s2_ref, gv_ref, bv_ref, gk_ref,
               bk_ref, m24_ref, s8_ref, vals_ref, frac_ref, base_ref,
               ksum_ref, ksq_ref):
    b = pl.program_id(0)
    i = pl.program_id(1)

    @pl.when(jnp.logical_and(b == 0, i == 0))
    def _():
        ksum_ref[...] = jnp.zeros_like(ksum_ref)
        ksq_ref[...] = jnp.zeros_like(ksq_ref)

    n = float(BATCH * NPTS)
    kv = jnp.dot(w_ref[...], x_ref[0], preferred_element_type=jnp.float32)

    mean = s1_ref[...] / n
    var = s2_ref[...] / n - mean * mean
    rstd = lax.rsqrt(var + 1e-5)

    ko = kv[0:NUM_HEADS * 3]
    vals = kv[NUM_HEADS * 3:]
    ko_n = (ko - mean[0:24]) * (rstd[0:24] * gk_ref[...]) + bk_ref[...]
    vals_n = (vals - mean[24:]) * (rstd[24:] * gv_ref[...]) + bv_ref[...]
    vals_ref[0] = vals_n

    orig = o_ref[0]
    pts = ko_n + jnp.concatenate([orig] * NUM_HEADS, axis=0)
    keys = jnp.dot(m24_ref[...], pts, preferred_element_type=jnp.float32)

    ksum_ref[...] += jnp.full((1, 128), jnp.sum(keys), jnp.float32)
    ksq_ref[...] += jnp.full((1, 128), jnp.sum(keys * keys), jnp.float32)

    lattice = jnp.tanh(keys)
    pos = (lattice + 1.0) * 0.5 * float(GRID_SIZE - 1)
    basef = jnp.clip(jnp.floor(pos), 0.0, float(GRID_SIZE - 2))
    frac_ref[0] = pos - basef
    flat = jnp.dot(s8_ref[...], basef, preferred_element_type=jnp.float32)
    base_ref[0] = flat.astype(jnp.int32)


def _splat_sc_body(vals_hbm, frac_hbm, base_hbm, zeros_hbm, out_hbm,
                   table_v, vals_a, frac_a, base_a, vals_b, frac_b, base_b,
                   sem_a, sem_b):
    wid = lax.axis_index("s") * SC_CORES + lax.axis_index("c")
    pair = wid

    def _copies(r, ci, vv, fv, bv, sem):
        c0 = ci * SC_CHUNK
        return (
            pltpu.make_async_copy(
                vals_hbm.at[pair, pl.ds(r * FHALF, FHALF),
                            pl.ds(c0, SC_CHUNK)], vv, sem),
            pltpu.make_async_copy(
                frac_hbm.at[pair, :, pl.ds(c0, SC_CHUNK)], fv, sem),
            pltpu.make_async_copy(
                base_hbm.at[pair, pl.ds(c0, SC_CHUNK)], bv, sem),
        )

    def _start(r, ci, vv, fv, bv, sem):
        for d in _copies(r, ci, vv, fv, bv, sem):
            d.start()

    def _wait(r, ci, vv, fv, bv, sem):
        for d in _copies(r, ci, vv, fv, bv, sem):
            d.wait()

    def _scatter_chunk(vv, fv, bv):
        def group_body(g, _):
            g0 = g * 16
            fx = fv[0, pl.ds(g0, 16)]
            fy = fv[1, pl.ds(g0, 16)]
            fz = fv[2, pl.ds(g0, 16)]
            bse = bv[pl.ds(g0, 16)]
            gx = 1.0 - fx
            gy = 1.0 - fy
            gz = 1.0 - fz
            w00 = gx * gy
            w01 = gx * fy
            w10 = fx * gy
            w11 = fx * fy
            wc = (w00 * gz, w00 * fz, w01 * gz, w01 * fz,
                  w10 * gz, w10 * fz, w11 * gz, w11 * fz)
            cells = tuple(bse + off for off in _CORNER_OFF)
            for f in range(FHALF):
                v = vv[f, pl.ds(g0, 16)]
                fbase = f * G3
                for c in range(8):
                    plsc.addupdate_scatter(
                        table_v, [cells[c] + fbase], wc[c] * v)
            return 0

        lax.fori_loop(0, SC_CHUNK // 16, group_body, 0)

    for r in range(N_ROUND):
        pltpu.sync_copy(zeros_hbm, table_v)
        _start(r, 0, vals_a, frac_a, base_a, sem_a)
        _start(r, 1, vals_b, frac_b, base_b, sem_b)

        def pair_body(i2, _, r=r):
            ca = 2 * i2
            _wait(r, ca, vals_a, frac_a, base_a, sem_a)
            _scatter_chunk(vals_a, frac_a, base_a)

            @pl.when(i2 < N_CHUNK // 2 - 1)
            def _():
                _start(r, ca + 2, vals_a, frac_a, base_a, sem_a)

            _wait(r, ca + 1, vals_b, frac_b, base_b, sem_b)
            _scatter_chunk(vals_b, frac_b, base_b)

            @pl.when(i2 < N_CHUNK // 2 - 1)
            def _():
                _start(r, ca + 3, vals_b, frac_b, base_b, sem_b)

            return 0

        lax.fori_loop(0, N_CHUNK // 2, pair_body, 0)
        pltpu.sync_copy(table_v, out_hbm.at[pair, r])


def _finalize_body(splat_ref, ksum_ref, ksq_ref, occ_ref, km_ref, kv_ref,
                   acc_ref):
    i = pl.program_id(0)

    @pl.when(i == 0)
    def _():
        acc_ref[...] = jnp.zeros_like(acc_ref)

    x = splat_ref[0]
    nz = (jnp.abs(x) > 1e-9).astype(jnp.float32)
    acc_ref[...] += jnp.sum(nz, axis=0, keepdims=True)

    @pl.when(i == pl.num_programs(0) - 1)
    def _():
        occ = jnp.sum(acc_ref[...]) / float(BATCH * FEATURE_DIM * NUM_HEADS)
        nk = float(BATCH * NUM_HEADS * 3 * NPTS)
        kmean = ksum_ref[0, 0] / nk
        kvar = (ksq_ref[0, 0] - nk * kmean * kmean) / (nk - 1.0)
        occ_ref[...] = jnp.full((1, 1), occ, jnp.float32)
        km_ref[...] = jnp.full((1, 1), kmean, jnp.float32)
        kv_ref[...] = jnp.full((1, 1), kvar, jnp.float32)


@functools.lru_cache(maxsize=1)
def _get_splat_sc():
    return pl.kernel(
        _splat_sc_body,
        out_type=jax.ShapeDtypeStruct((NPAIR, N_ROUND, TBL), jnp.float32),
        mesh=plsc.VectorSubcoreMesh(core_axis_name="c", subcore_axis_name="s"),
        compiler_params=pltpu.CompilerParams(needs_layout_passes=False),
        scratch_types=[
            pltpu.VMEM((TBL,), jnp.float32),
            pltpu.VMEM((FHALF, SC_CHUNK), jnp.float32),
            pltpu.VMEM((3, SC_CHUNK), jnp.float32),
            pltpu.VMEM((SC_CHUNK,), jnp.int32),
            pltpu.VMEM((FHALF, SC_CHUNK), jnp.float32),
            pltpu.VMEM((3, SC_CHUNK), jnp.float32),
            pltpu.VMEM((SC_CHUNK,), jnp.int32),
            pltpu.SemaphoreType.DMA,
            pltpu.SemaphoreType.DMA,
        ],
    )


def kernel(input_tensor, original_points, W_kv, values_gamma, values_beta,
           key_gamma, key_beta, rot):
    f32 = jnp.float32

    s1, s2 = pl.pallas_call(
        _stats_body,
        grid=(BATCH, NSTEP),
        in_specs=[
            pl.BlockSpec((1, MODEL_DIM, PT), lambda b, i: (b, 0, i)),
            pl.BlockSpec((C_OUT, MODEL_DIM), lambda b, i: (0, 0)),
        ],
        out_specs=[
            pl.BlockSpec((C_OUT, 1), lambda b, i: (0, 0)),
            pl.BlockSpec((C_OUT, 1), lambda b, i: (0, 0)),
        ],
        out_shape=[
            jax.ShapeDtypeStruct((C_OUT, 1), f32),
            jax.ShapeDtypeStruct((C_OUT, 1), f32),
        ],
    )(input_tensor, W_kv)

    m24 = jax.scipy.linalg.block_diag(*[rot[h] for h in range(NUM_HEADS)])
    m24 = m24.astype(f32)
    s8 = jnp.zeros((NUM_HEADS, NUM_HEADS * 3), f32)
    scales = jnp.array([GRID_SIZE * GRID_SIZE, GRID_SIZE, 1], f32)
    rows = jnp.repeat(jnp.arange(NUM_HEADS), 3)
    cols = jnp.arange(NUM_HEADS * 3)
    s8 = s8.at[rows, cols].set(jnp.tile(scales, NUM_HEADS))

    vals_n, frac24, basef, ksum, ksq = pl.pallas_call(
        _geom_body,
        grid=(BATCH, NSTEP),
        in_specs=[
            pl.BlockSpec((1, MODEL_DIM, PT), lambda b, i: (b, 0, i)),
            pl.BlockSpec((1, 3, PT), lambda b, i: (b, 0, i)),
            pl.BlockSpec((C_OUT, MODEL_DIM), lambda b, i: (0, 0)),
            pl.BlockSpec((C_OUT, 1), lambda b, i: (0, 0)),
            pl.BlockSpec((C_OUT, 1), lambda b, i: (0, 0)),
            pl.BlockSpec((NUM_HEADS * FEATURE_DIM, 1), lambda b, i: (0, 0)),
            pl.BlockSpec((NUM_HEADS * FEATURE_DIM, 1), lambda b, i: (0, 0)),
            pl.BlockSpec((NUM_HEADS * 3, 1), lambda b, i: (0, 0)),
            pl.BlockSpec((NUM_HEADS * 3, 1), lambda b, i: (0, 0)),
            pl.BlockSpec((24, 24), lambda b, i: (0, 0)),
            pl.BlockSpec((NUM_HEADS, 24), lambda b, i: (0, 0)),
        ],
        out_specs=[
            pl.BlockSpec((1, NUM_HEADS * FEATURE_DIM, PT), lambda b, i: (b, 0, i)),
            pl.BlockSpec((1, NUM_HEADS * 3, PT), lambda b, i: (b, 0, i)),
            pl.BlockSpec((1, NUM_HEADS, PT), lambda b, i: (b, 0, i)),
            pl.BlockSpec((1, 128), lambda b, i: (0, 0)),
            pl.BlockSpec((1, 128), lambda b, i: (0, 0)),
        ],
        out_shape=[
            jax.ShapeDtypeStruct((BATCH, NUM_HEADS * FEATURE_DIM, NPTS), f32),
            jax.ShapeDtypeStruct((BATCH, NUM_HEADS * 3, NPTS), f32),
            jax.ShapeDtypeStruct((BATCH, NUM_HEADS, NPTS), jnp.int32),
            jax.ShapeDtypeStruct((1, 128), f32),
            jax.ShapeDtypeStruct((1, 128), f32),
        ],
    )(input_tensor, original_points, W_kv, s1, s2,
      values_gamma.reshape(-1, 1), values_beta.reshape(-1, 1),
      key_gamma.reshape(-1, 1), key_beta.reshape(-1, 1), m24, s8)

    vals_sc = vals_n.reshape(NPAIR, FEATURE_DIM, NPTS)
    frac_sc = frac24.reshape(NPAIR, 3, NPTS)
    base_sc = basef.reshape(NPAIR, NPTS)
    zeros_tbl = jnp.zeros((TBL,), f32)
    splat_flat = _get_splat_sc()(vals_sc, frac_sc, base_sc, zeros_tbl)

    splat_view = splat_flat.reshape(NPAIR * N_ROUND, TBL // 128, 128)
    occ, kmean, kvar = pl.pallas_call(
        _finalize_body,
        grid=(NPAIR * N_ROUND,),
        in_specs=[
            pl.BlockSpec((1, TBL // 128, 128), lambda i: (i, 0, 0)),
            pl.BlockSpec((1, 128), lambda i: (0, 0)),
            pl.BlockSpec((1, 128), lambda i: (0, 0)),
        ],
        out_specs=[
            pl.BlockSpec((1, 1), lambda i: (0, 0)),
            pl.BlockSpec((1, 1), lambda i: (0, 0)),
            pl.BlockSpec((1, 1), lambda i: (0, 0)),
        ],
        out_shape=[
            jax.ShapeDtypeStruct((1, 1), f32),
            jax.ShapeDtypeStruct((1, 1), f32),
            jax.ShapeDtypeStruct((1, 1), f32),
        ],
        scratch_shapes=[pltpu.VMEM((1, 128), f32)],
    )(splat_view, ksum, ksq)

    splat = splat_flat.reshape(BATCH, NUM_HEADS * FEATURE_DIM,
                               GRID_SIZE, GRID_SIZE, GRID_SIZE)
    return (splat, occ[0, 0], kmean[0, 0], kvar[0, 0])


# speedup vs baseline: 1.1316x; 1.0047x over previous
import functools

import jax
import jax.numpy as jnp
from jax import lax
from jax.experimental import pallas as pl
from jax.experimental.pallas import tpu as pltpu
from jax.experimental.pallas import tpu_sc as plsc

MODEL_DIM = 256
FEATURE_DIM = 32
GRID_SIZE = 16
NUM_HEADS = 8
BATCH = 4
NPTS = 8192
C_OUT = NUM_HEADS * (FEATURE_DIM + 3)
G3 = GRID_SIZE ** 3
NPAIR = BATCH * NUM_HEADS

PT = 8192
NSTEP = NPTS // PT

SC_CORES = 2
SC_SUBCORES = 16
NW = SC_CORES * SC_SUBCORES

SC_CHUNK = 1024
N_CHUNK = NPTS // SC_CHUNK
FHALF = 16
N_ROUND = FEATURE_DIM // FHALF
TBL = FHALF * G3

_CORNER_OFF = [
    (bx * GRID_SIZE * GRID_SIZE + by * GRID_SIZE + bz)
    for bx in (0, 1) for by in (0, 1) for bz in (0, 1)
]


def _stats_body(x_ref, w_ref, s1_ref, s2_ref):
    b = pl.program_id(0)
    i = pl.program_id(1)

    @pl.when(jnp.logical_and(b == 0, i == 0))
    def _():
        s1_ref[...] = jnp.zeros_like(s1_ref)
        s2_ref[...] = jnp.zeros_like(s2_ref)

    kv = jnp.dot(w_ref[...], x_ref[0], preferred_element_type=jnp.float32)
    s1_ref[...] += jnp.sum(kv, axis=1, keepdims=True)
    s2_ref[...] += jnp.sum(kv * kv, axis=1, keepdims=True)


def _geom_body(x_ref, o_ref, w_ref, s1_ref, s2_ref, gv_ref, bv_ref, gk_ref,
               bk_ref, m24_ref, s8_ref, vals_ref, frac_ref, base_ref,
               ksum_ref, ksq_ref):
    b = pl.program_id(0)
    i = pl.program_id(1)

    @pl.when(jnp.logical_and(b == 0, i == 0))
    def _():
        ksum_ref[...] = jnp.zeros_like(ksum_ref)
        ksq_ref[...] = jnp.zeros_like(ksq_ref)

    n = float(BATCH * NPTS)
    kv = jnp.dot(w_ref[...], x_ref[0], preferred_element_type=jnp.float32)

    mean = s1_ref[...] / n
    var = s2_ref[...] / n - mean * mean
    rstd = lax.rsqrt(var + 1e-5)

    ko = kv[0:NUM_HEADS * 3]
    vals = kv[NUM_HEADS * 3:]
    ko_n = (ko - mean[0:24]) * (rstd[0:24] * gk_ref[...]) + bk_ref[...]
    vals_n = (vals - mean[24:]) * (rstd[24:] * gv_ref[...]) + bv_ref[...]
    vals_ref[0] = vals_n

    orig = o_ref[0]
    pts = ko_n + jnp.concatenate([orig] * NUM_HEADS, axis=0)
    keys = jnp.dot(m24_ref[...], pts, preferred_element_type=jnp.float32)

    ksum_ref[...] += jnp.full((1, 128), jnp.sum(keys), jnp.float32)
    ksq_ref[...] += jnp.full((1, 128), jnp.sum(keys * keys), jnp.float32)

    lattice = jnp.tanh(keys)
    pos = (lattice + 1.0) * 0.5 * float(GRID_SIZE - 1)
    basef = jnp.clip(jnp.floor(pos), 0.0, float(GRID_SIZE - 2))
    frac_ref[0] = pos - basef
    flat = jnp.dot(s8_ref[...], basef, preferred_element_type=jnp.float32)
    base_ref[0] = flat.astype(jnp.int32)


def _splat_sc_body(vals_hbm, frac_hbm, base_hbm, zeros_hbm, out_hbm,
                   table_v, vals_a, frac_a, base_a, vals_b, frac_b, base_b,
                   sem_a, sem_b):
    wid = lax.axis_index("s") * SC_CORES + lax.axis_index("c")
    pair = wid

    def _copies(r, ci, vv, fv, bv, sem):
        c0 = ci * SC_CHUNK
        return (
            pltpu.make_async_copy(
                vals_hbm.at[pair, pl.ds(r * FHALF, FHALF),
                            pl.ds(c0, SC_CHUNK)], vv, sem),
            pltpu.make_async_copy(
                frac_hbm.at[pair, :, pl.ds(c0, SC_CHUNK)], fv, sem),
            pltpu.make_async_copy(
                base_hbm.at[pair, pl.ds(c0, SC_CHUNK)], bv, sem),
        )

    def _start(r, ci, vv, fv, bv, sem):
        for d in _copies(r, ci, vv, fv, bv, sem):
            d.start()

    def _wait(r, ci, vv, fv, bv, sem):
        for d in _copies(r, ci, vv, fv, bv, sem):
            d.wait()

    def _scatter_chunk(vv, fv, bv):
        def group_body(g, _):
            g0 = g * 16
            fx = fv[0, pl.ds(g0, 16)]
            fy = fv[1, pl.ds(g0, 16)]
            fz = fv[2, pl.ds(g0, 16)]
            bse = bv[pl.ds(g0, 16)]
            gx = 1.0 - fx
            gy = 1.0 - fy
            gz = 1.0 - fz
            w00 = gx * gy
            w01 = gx * fy
            w10 = fx * gy
            w11 = fx * fy
            wc = (w00 * gz, w00 * fz, w01 * gz, w01 * fz,
                  w10 * gz, w10 * fz, w11 * gz, w11 * fz)
            cells = tuple(bse + off for off in _CORNER_OFF)
            for f in range(FHALF):
                v = vv[f, pl.ds(g0, 16)]
                fbase = f * G3
                for c in range(8):
                    plsc.addupdate_scatter(
                        table_v, [cells[c] + fbase], wc[c] * v)
            return 0

        lax.fori_loop(0, SC_CHUNK // 16, group_body, 0)

    for r in range(N_ROUND):
        pltpu.sync_copy(zeros_hbm, table_v)
        _start(r, 0, vals_a, frac_a, base_a, sem_a)
        _start(r, 1, vals_b, frac_b, base_b, sem_b)

        def pair_body(i2, _, r=r):
            ca = 2 * i2
            _wait(r, ca, vals_a, frac_a, base_a, sem_a)
            _scatter_chunk(vals_a, frac_a, base_a)

            @pl.when(i2 < N_CHUNK // 2 - 1)
            def _():
                _start(r, ca + 2, vals_a, frac_a, base_a, sem_a)

            _wait(r, ca + 1, vals_b, frac_b, base_b, sem_b)
            _scatter_chunk(vals_b, frac_b, base_b)

            @pl.when(i2 < N_CHUNK // 2 - 1)
            def _():
                _start(r, ca + 3, vals_b, frac_b, base_b, sem_b)

            return 0

        lax.fori_loop(0, N_CHUNK // 2, pair_body, 0)
        pltpu.sync_copy(table_v, out_hbm.at[pair, r])


def _finalize_body(splat_ref, ksum_ref, ksq_ref, occ_ref, km_ref, kv_ref,
                   acc_ref):
    i = pl.program_id(0)

    @pl.when(i == 0)
    def _():
        acc_ref[...] = jnp.zeros_like(acc_ref)

    x = splat_ref[0]
    nz = (jnp.abs(x) > 1e-9).astype(jnp.float32)
    acc_ref[...] += jnp.sum(nz, axis=0, keepdims=True)

    @pl.when(i == pl.num_programs(0) - 1)
    def _():
        occ = jnp.sum(acc_ref[...]) / float(BATCH * FEATURE_DIM * NUM_HEADS)
        nk = float(BATCH * NUM_HEADS * 3 * NPTS)
        kmean = ksum_ref[0, 0] / nk
        kvar = (ksq_ref[0, 0] - nk * kmean * kmean) / (nk - 1.0)
        occ_ref[...] = jnp.full((1, 1), occ, jnp.float32)
        km_ref[...] = jnp.full((1, 1), kmean, jnp.float32)
        kv_ref[...] = jnp.full((1, 1), kvar, jnp.float32)


@functools.lru_cache(maxsize=1)
def _get_splat_sc():
    return pl.kernel(
        _splat_sc_body,
        out_type=jax.ShapeDtypeStruct((NPAIR, N_ROUND, TBL), jnp.float32),
        mesh=plsc.VectorSubcoreMesh(core_axis_name="c", subcore_axis_name="s"),
        compiler_params=pltpu.CompilerParams(needs_layout_passes=False),
        scratch_types=[
            pltpu.VMEM((TBL,), jnp.float32),
            pltpu.VMEM((FHALF, SC_CHUNK), jnp.float32),
            pltpu.VMEM((3, SC_CHUNK), jnp.float32),
            pltpu.VMEM((SC_CHUNK,), jnp.int32),
            pltpu.VMEM((FHALF, SC_CHUNK), jnp.float32),
            pltpu.VMEM((3, SC_CHUNK), jnp.float32),
            pltpu.VMEM((SC_CHUNK,), jnp.int32),
            pltpu.SemaphoreType.DMA,
            pltpu.SemaphoreType.DMA,
        ],
    )


def kernel(input_tensor, original_points, W_kv, values_gamma, values_beta,
           key_gamma, key_beta, rot):
    f32 = jnp.float32

    s1, s2 = pl.pallas_call(
        _stats_body,
        grid=(BATCH, NSTEP),
        in_specs=[
            pl.BlockSpec((1, MODEL_DIM, PT), lambda b, i: (b, 0, i)),
            pl.BlockSpec((C_OUT, MODEL_DIM), lambda b, i: (0, 0)),
        ],
        out_specs=[
            pl.BlockSpec((C_OUT, 1), lambda b, i: (0, 0)),
            pl.BlockSpec((C_OUT, 1), lambda b, i: (0, 0)),
        ],
        out_shape=[
            jax.ShapeDtypeStruct((C_OUT, 1), f32),
            jax.ShapeDtypeStruct((C_OUT, 1), f32),
        ],
    )(input_tensor, W_kv)

    m24 = jax.scipy.linalg.block_diag(*[rot[h] for h in range(NUM_HEADS)])
    m24 = m24.astype(f32)
    s8 = jnp.zeros((NUM_HEADS, NUM_HEADS * 3), f32)
    scales = jnp.array([GRID_SIZE * GRID_SIZE, GRID_SIZE, 1], f32)
    rows = jnp.repeat(jnp.arange(NUM_HEADS), 3)
    cols = jnp.arange(NUM_HEADS * 3)
    s8 = s8.at[rows, cols].set(jnp.tile(scales, NUM_HEADS))

    vals_n, frac24, basef, ksum, ksq = pl.pallas_call(
        _geom_body,
        grid=(BATCH, NSTEP),
        in_specs=[
            pl.BlockSpec((1, MODEL_DIM, PT), lambda b, i: (b, 0, i)),
            pl.BlockSpec((1, 3, PT), lambda b, i: (b, 0, i)),
            pl.BlockSpec((C_OUT, MODEL_DIM), lambda b, i: (0, 0)),
            pl.BlockSpec((C_OUT, 1), lambda b, i: (0, 0)),
            pl.BlockSpec((C_OUT, 1), lambda b, i: (0, 0)),
            pl.BlockSpec((NUM_HEADS * FEATURE_DIM, 1), lambda b, i: (0, 0)),
            pl.BlockSpec((NUM_HEADS * FEATURE_DIM, 1), lambda b, i: (0, 0)),
            pl.BlockSpec((NUM_HEADS * 3, 1), lambda b, i: (0, 0)),
            pl.BlockSpec((NUM_HEADS * 3, 1), lambda b, i: (0, 0)),
            pl.BlockSpec((24, 24), lambda b, i: (0, 0)),
            pl.BlockSpec((NUM_HEADS, 24), lambda b, i: (0, 0)),
        ],
        out_specs=[
            pl.BlockSpec((1, NUM_HEADS * FEATURE_DIM, PT), lambda b, i: (b, 0, i)),
            pl.BlockSpec((1, NUM_HEADS * 3, PT), lambda b, i: (b, 0, i)),
            pl.BlockSpec((1, NUM_HEADS, PT), lambda b, i: (b, 0, i)),
            pl.BlockSpec((1, 128), lambda b, i: (0, 0)),
            pl.BlockSpec((1, 128), lambda b, i: (0, 0)),
        ],
        out_shape=[
            jax.ShapeDtypeStruct((BATCH, NUM_HEADS * FEATURE_DIM, NPTS), f32),
            jax.ShapeDtypeStruct((BATCH, NUM_HEADS * 3, NPTS), f32),
            jax.ShapeDtypeStruct((BATCH, NUM_HEADS, NPTS), jnp.int32),
            jax.ShapeDtypeStruct((1, 128), f32),
            jax.ShapeDtypeStruct((1, 128), f32),
        ],
    )(input_tensor, original_points, W_kv, s1, s2,
      values_gamma.reshape(-1, 1), values_beta.reshape(-1, 1),
      key_gamma.reshape(-1, 1), key_beta.reshape(-1, 1), m24, s8)

    vals_sc = vals_n.reshape(NPAIR, FEATURE_DIM, NPTS)
    frac_sc = frac24.reshape(NPAIR, 3, NPTS)
    base_sc = basef.reshape(NPAIR, NPTS)
    zeros_tbl = jnp.zeros((TBL,), f32)
    splat_flat = _get_splat_sc()(vals_sc, frac_sc, base_sc, zeros_tbl)

    splat_view = splat_flat.reshape(NPAIR * N_ROUND, TBL // 128, 128)
    occ, kmean, kvar = pl.pallas_call(
        _finalize_body,
        grid=(NPAIR * N_ROUND,),
        in_specs=[
            pl.BlockSpec((1, TBL // 128, 128), lambda i: (i, 0, 0)),
            pl.BlockSpec((1, 128), lambda i: (0, 0)),
            pl.BlockSpec((1, 128), lambda i: (0, 0)),
        ],
        out_specs=[
            pl.BlockSpec((1, 1), lambda i: (0, 0)),
            pl.BlockSpec((1, 1), lambda i: (0, 0)),
            pl.BlockSpec((1, 1), lambda i: (0, 0)),
        ],
        out_shape=[
            jax.ShapeDtypeStruct((1, 1), f32),
            jax.ShapeDtypeStruct((1, 1), f32),
            jax.ShapeDtypeStruct((1, 1), f32),
        ],
        scratch_shapes=[pltpu.VMEM((1, 128), f32)],
    )(splat_view, ksum, ksq)

    splat = splat_flat.reshape(BATCH, NUM_HEADS * FEATURE_DIM,
                               GRID_SIZE, GRID_SIZE, GRID_SIZE)
    return (splat, occ[0, 0], kmean[0, 0], kvar[0, 0])
